# baseline (speedup 1.0000x reference)
import jax
import jax.numpy as jnp
from jax.experimental import pallas as pl


def _pe_add_kernel(x_ref, pe_ref, o_ref):
    L = x_ref.shape[1]
    o_ref[...] = x_ref[...] + pe_ref[:L, :][None, :, :]


def kernel(x, pe):
    B, L, D = x.shape
    BB = 256
    grid = (B // BB,)
    return pl.pallas_call(
        _pe_add_kernel,
        grid=grid,
        in_specs=[
            pl.BlockSpec((BB, L, D), lambda i: (i, 0, 0)),
            pl.BlockSpec(pe.shape, lambda i: (0, 0)),
        ],
        out_specs=pl.BlockSpec((BB, L, D), lambda i: (i, 0, 0)),
        out_shape=jax.ShapeDtypeStruct((B, L, D), x.dtype),
    )(x, pe)
